# Initial kernel scaffold; baseline (speedup 1.0000x reference)
import jax
import jax.numpy as jnp
from jax.experimental import pallas as pl
from jax.experimental.pallas import tpu as pltpu

B = 64
DIN = 256
H = 128
KDIM = 128
DICT = 100000
A2CH = 128
NA = 6

TILE = 2000
NT = DICT // TILE


def _k1_body(q_ref, idc_ref, idr_ref, keys_ref, vals_ref,
             nk_ref, nv_ref, best_ref, qp_ref,
             bv_scr, bi_scr):
    g = pl.program_id(0)

    k_tile = keys_ref[...]
    nk_ref[...] = k_tile
    nv_ref[...] = vals_ref[...]

    q = q_ref[...]
    qn = q / (jnp.sqrt(jnp.sum(q * q, axis=1, keepdims=True)) + 1e-8)
    kn = k_tile / (jnp.sqrt(jnp.sum(k_tile * k_tile, axis=1, keepdims=True)) + 1e-8)
    s = jax.lax.dot_general(qn, kn, (((1,), (1,)), ((), ())),
                            preferred_element_type=jnp.float32)

    lmax = jnp.max(s, axis=1, keepdims=True)
    li = jax.lax.broadcasted_iota(jnp.int32, s.shape, 1)
    lidx = jnp.min(jnp.where(s == lmax, li, TILE), axis=1, keepdims=True) + g * TILE

    @pl.when(g == 0)
    def _init():
        bv_scr[...] = jnp.full((B, 1), -jnp.inf, jnp.float32)
        bi_scr[...] = jnp.zeros((B, 1), jnp.int32)
        idc = idc_ref[...]
        idr = idr_ref[...]
        jj = jax.lax.broadcasted_iota(jnp.int32, (B, B), 1)
        eq = idc == idr
        last = jnp.max(jnp.where(eq, jj, -1), axis=1, keepdims=True)
        P = (jj == last).astype(jnp.float32)
        qp_ref[...] = jax.lax.dot_general(P, q, (((1,), (0,)), ((), ())),
                                          preferred_element_type=jnp.float32)

    upd = lmax > bv_scr[...]
    bv_scr[...] = jnp.where(upd, lmax, bv_scr[...])
    bi_scr[...] = jnp.where(upd, lidx, bi_scr[...])

    @pl.when(g == NT - 1)
    def _fin():
        best_ref[...] = bi_scr[...]


def _k2_body(best_ref, bid_ref, x_ref, h_ref, c_ref, Wi_ref, bi_ref,
             Wh_ref, bh_ref, W1_ref, b1_ref, Wa_ref, ba_ref, Wc_ref, bc_ref,
             idc_ref, idr_ref, qp_ref, vals_any, nk_in, nv_in,
             a_ref, prob_ref, v_ref, ent_ref, ht_ref, ct_ref,
             nk_out, nv_out,
             mem_scr, cp_scr, sem):
    def dotT(a, w):
        return jax.lax.dot_general(a, w, (((1,), (1,)), ((), ())),
                                   preferred_element_type=jnp.float32)

    for b in range(B):
        idx = best_ref[b]
        pltpu.make_async_copy(vals_any.at[pl.ds(idx, 1), :],
                              mem_scr.at[pl.ds(b, 1), :], sem).start()

    x = x_ref[...]
    hh = h_ref[...]
    preact = (dotT(x, Wi_ref[...]) + bi_ref[...]
              + dotT(hh, Wh_ref[...]) + bh_ref[...])
    gates = jax.nn.sigmoid(preact[:, :4 * H])
    f_t = gates[:, 0:H]
    i_t = gates[:, H:2 * H]
    o_t = gates[:, 2 * H:3 * H]
    r_t = gates[:, 3 * H:4 * H]
    c_new = jnp.tanh(preact[:, 4 * H:5 * H])
    c_t = f_t * c_ref[...] + i_t * c_new

    for b in range(B):
        idx = best_ref[b]
        pltpu.make_async_copy(vals_any.at[pl.ds(idx, 1), :],
                              mem_scr.at[pl.ds(b, 1), :], sem).wait()

    m_t = jnp.tanh(mem_scr[...])
    c_t = c_t + r_t * m_t
    h_t = o_t * jnp.tanh(c_t)
    ht_ref[...] = h_t
    ct_ref[...] = c_t

    hid = jax.nn.relu(dotT(c_t, W1_ref[...]) + b1_ref[...])
    logits = dotT(hid, Wa_ref[...]) + ba_ref[...]
    lm = jnp.max(logits, axis=1, keepdims=True)
    e = jnp.exp(logits - lm)
    pi = e / jnp.sum(e, axis=1, keepdims=True)
    ent_ref[...] = -jnp.sum(pi * jnp.log(pi + 1e-12), axis=1, keepdims=True)
    pmax = jnp.max(pi, axis=1, keepdims=True)
    ai = jax.lax.broadcasted_iota(jnp.int32, pi.shape, 1)
    a_ref[...] = jnp.min(jnp.where(pi == pmax, ai, NA), axis=1, keepdims=True)
    prob_ref[...] = jnp.log(pmax + 1e-12)
    v_ref[...] = dotT(hid, Wc_ref[...]) + bc_ref[...]

    idc = idc_ref[...]
    idr = idr_ref[...]
    jj = jax.lax.broadcasted_iota(jnp.int32, (B, B), 1)
    eq = idc == idr
    last = jnp.max(jnp.where(eq, jj, -1), axis=1, keepdims=True)
    P = (jj == last).astype(jnp.float32)
    cp_scr[...] = jax.lax.dot_general(P, c_t, (((1,), (0,)), ((), ())),
                                      preferred_element_type=jnp.float32)

    for b in range(B):
        sid = bid_ref[b]
        pltpu.make_async_copy(qp_ref.at[pl.ds(b, 1), :],
                              nk_out.at[pl.ds(sid, 1), :], sem).start()
        pltpu.make_async_copy(cp_scr.at[pl.ds(b, 1), :],
                              nv_out.at[pl.ds(sid, 1), :], sem).start()
    for b in range(B):
        sid = bid_ref[b]
        pltpu.make_async_copy(qp_ref.at[pl.ds(b, 1), :],
                              nk_out.at[pl.ds(sid, 1), :], sem).wait()
        pltpu.make_async_copy(cp_scr.at[pl.ds(b, 1), :],
                              nv_out.at[pl.ds(sid, 1), :], sem).wait()


def kernel(obs_bar_reward, barcode_tensor, barcode_id, h, c, Wi, bi, Wh, bh,
           dnd_keys, dnd_vals, W1, b1, Wa, ba, Wc, bc):
    f32 = jnp.float32
    idc = barcode_id.reshape(B, 1)
    idr = barcode_id.reshape(1, B)

    vmem = lambda shape: pl.BlockSpec(shape, lambda g: (0, 0))
    tile = lambda w: pl.BlockSpec((TILE, w), lambda g: (g, 0))

    nk_pre, nv_pre, best_id, q_payload = pl.pallas_call(
        _k1_body,
        grid=(NT,),
        in_specs=[vmem((B, KDIM)), vmem((B, 1)), vmem((1, B)),
                  tile(KDIM), tile(H)],
        out_specs=[tile(KDIM), tile(H), vmem((B, 1)), vmem((B, KDIM))],
        out_shape=[jax.ShapeDtypeStruct((DICT, KDIM), f32),
                   jax.ShapeDtypeStruct((DICT, H), f32),
                   jax.ShapeDtypeStruct((B, 1), jnp.int32),
                   jax.ShapeDtypeStruct((B, KDIM), f32)],
        scratch_shapes=[pltpu.VMEM((B, 1), f32), pltpu.VMEM((B, 1), jnp.int32)],
    )(barcode_tensor, idc, idr, dnd_keys, dnd_vals)

    smem1d = pl.BlockSpec(memory_space=pltpu.SMEM)
    anyspec = pl.BlockSpec(memory_space=pltpu.ANY)
    vfull = pl.BlockSpec(memory_space=pltpu.VMEM)

    outs = pl.pallas_call(
        _k2_body,
        in_specs=[smem1d, smem1d, vfull, vfull, vfull, vfull, vfull,
                  vfull, vfull, vfull, vfull, vfull, vfull, vfull, vfull,
                  vfull, vfull, vfull, anyspec, anyspec, anyspec],
        out_specs=[vfull, vfull, vfull, vfull, vfull, vfull, anyspec, anyspec],
        out_shape=[jax.ShapeDtypeStruct((B, 1), jnp.int32),
                   jax.ShapeDtypeStruct((B, 1), f32),
                   jax.ShapeDtypeStruct((B, 1), f32),
                   jax.ShapeDtypeStruct((B, 1), f32),
                   jax.ShapeDtypeStruct((B, H), f32),
                   jax.ShapeDtypeStruct((B, H), f32),
                   jax.ShapeDtypeStruct((DICT, KDIM), f32),
                   jax.ShapeDtypeStruct((DICT, H), f32)],
        scratch_shapes=[pltpu.VMEM((B, H), f32), pltpu.VMEM((B, H), f32),
                        pltpu.SemaphoreType.DMA],
        input_output_aliases={19: 6, 20: 7},
    )(best_id.reshape(B), barcode_id, obs_bar_reward, h, c,
      Wi, bi.reshape(1, 5 * H), Wh, bh.reshape(1, 5 * H),
      W1, b1.reshape(1, A2CH), Wa, ba.reshape(1, NA), Wc, bc.reshape(1, 1),
      idc, idr, q_payload, dnd_vals, nk_pre, nv_pre)

    a_t, prob_a_t, v_t, entropy, h_t, c_t, new_keys, new_vals = outs
    return (a_t.reshape(B), prob_a_t.reshape(B), v_t, entropy.reshape(B),
            h_t, c_t, best_id.reshape(B), new_keys, new_vals)


# speedup vs baseline: 2.0411x; 2.0411x over previous
import jax
import jax.numpy as jnp
from jax.experimental import pallas as pl
from jax.experimental.pallas import tpu as pltpu

B = 64
DIN = 256
H = 128
KDIM = 128
DICT = 100000
A2CH = 128
NA = 6

TILE = 2000
NT = DICT // TILE


def _k1_body(q_ref, idc_ref, idr_ref, keys_ref, vals_ref,
             nk_ref, nv_ref, best_ref, qp_ref,
             bv_scr, bi_scr):
    g = pl.program_id(0)

    k_tile = keys_ref[...]
    nk_ref[...] = k_tile
    nv_ref[...] = vals_ref[...]

    q = q_ref[...]
    qn = q / (jnp.sqrt(jnp.sum(q * q, axis=1, keepdims=True)) + 1e-8)
    kn = k_tile / (jnp.sqrt(jnp.sum(k_tile * k_tile, axis=1, keepdims=True)) + 1e-8)
    s = jax.lax.dot_general(qn, kn, (((1,), (1,)), ((), ())),
                            preferred_element_type=jnp.float32)

    lmax = jnp.max(s, axis=1, keepdims=True)
    li = jax.lax.broadcasted_iota(jnp.int32, s.shape, 1)
    lidx = jnp.min(jnp.where(s == lmax, li, TILE), axis=1, keepdims=True) + g * TILE

    @pl.when(g == 0)
    def _init():
        bv_scr[...] = jnp.full((B, 1), -jnp.inf, jnp.float32)
        bi_scr[...] = jnp.zeros((B, 1), jnp.int32)
        idc = idc_ref[...]
        idr = idr_ref[...]
        jj = jax.lax.broadcasted_iota(jnp.int32, (B, B), 1)
        eq = idc == idr
        last = jnp.max(jnp.where(eq, jj, -1), axis=1, keepdims=True)
        P = (jj == last).astype(jnp.float32)
        qp_ref[...] = jax.lax.dot_general(P, q, (((1,), (0,)), ((), ())),
                                          preferred_element_type=jnp.float32)

    upd = lmax > bv_scr[...]
    bv_scr[...] = jnp.where(upd, lmax, bv_scr[...])
    bi_scr[...] = jnp.where(upd, lidx, bi_scr[...])

    @pl.when(g == NT - 1)
    def _fin():
        best_ref[...] = bi_scr[...]


def _k2_body(best_ref, bid_ref, x_ref, h_ref, c_ref, Wi_ref, bi_ref,
             Wh_ref, bh_ref, W1_ref, b1_ref, Wa_ref, ba_ref, Wc_ref, bc_ref,
             idc_ref, idr_ref, qp_ref, vals_any, nk_in, nv_in,
             a_ref, prob_ref, v_ref, ent_ref, ht_ref, ct_ref,
             nk_out, nv_out,
             mem_scr, cp_scr, sem):
    def dotT(a, w):
        return jax.lax.dot_general(a, w, (((1,), (1,)), ((), ())),
                                   preferred_element_type=jnp.float32)

    for b in range(B):
        idx = best_ref[b]
        pltpu.make_async_copy(vals_any.at[pl.ds(idx, 1), :],
                              mem_scr.at[pl.ds(b, 1), :], sem).start()

    x = x_ref[...]
    hh = h_ref[...]
    preact = (dotT(x, Wi_ref[...]) + bi_ref[...]
              + dotT(hh, Wh_ref[...]) + bh_ref[...])
    gates = jax.nn.sigmoid(preact[:, :4 * H])
    f_t = gates[:, 0:H]
    i_t = gates[:, H:2 * H]
    o_t = gates[:, 2 * H:3 * H]
    r_t = gates[:, 3 * H:4 * H]
    c_new = jnp.tanh(preact[:, 4 * H:5 * H])
    c_t = f_t * c_ref[...] + i_t * c_new

    for b in range(B):
        idx = best_ref[b]
        pltpu.make_async_copy(vals_any.at[pl.ds(idx, 1), :],
                              mem_scr.at[pl.ds(b, 1), :], sem).wait()

    m_t = jnp.tanh(mem_scr[...])
    c_t = c_t + r_t * m_t
    h_t = o_t * jnp.tanh(c_t)
    ht_ref[...] = h_t
    ct_ref[...] = c_t

    hid = jax.nn.relu(dotT(c_t, W1_ref[...]) + b1_ref[...])
    logits = dotT(hid, Wa_ref[...]) + ba_ref[...]
    lm = jnp.max(logits, axis=1, keepdims=True)
    e = jnp.exp(logits - lm)
    pi = e / jnp.sum(e, axis=1, keepdims=True)
    ent_ref[...] = -jnp.sum(pi * jnp.log(pi + 1e-12), axis=1, keepdims=True)
    pmax = jnp.max(pi, axis=1, keepdims=True)
    ai = jax.lax.broadcasted_iota(jnp.int32, pi.shape, 1)
    a_ref[...] = jnp.min(jnp.where(pi == pmax, ai, NA), axis=1, keepdims=True)
    prob_ref[...] = jnp.log(pmax + 1e-12)
    v_ref[...] = jnp.sum(hid * Wc_ref[...], axis=1, keepdims=True) + bc_ref[0]

    idc = idc_ref[...]
    idr = idr_ref[...]
    jj = jax.lax.broadcasted_iota(jnp.int32, (B, B), 1)
    eq = idc == idr
    last = jnp.max(jnp.where(eq, jj, -1), axis=1, keepdims=True)
    P = (jj == last).astype(jnp.float32)
    cp_scr[...] = jax.lax.dot_general(P, c_t, (((1,), (0,)), ((), ())),
                                      preferred_element_type=jnp.float32)

    for b in range(B):
        sid = bid_ref[b]
        pltpu.make_async_copy(qp_ref.at[pl.ds(b, 1), :],
                              nk_out.at[pl.ds(sid, 1), :], sem).start()
        pltpu.make_async_copy(cp_scr.at[pl.ds(b, 1), :],
                              nv_out.at[pl.ds(sid, 1), :], sem).start()
    for b in range(B):
        sid = bid_ref[b]
        pltpu.make_async_copy(qp_ref.at[pl.ds(b, 1), :],
                              nk_out.at[pl.ds(sid, 1), :], sem).wait()
        pltpu.make_async_copy(cp_scr.at[pl.ds(b, 1), :],
                              nv_out.at[pl.ds(sid, 1), :], sem).wait()


def kernel(obs_bar_reward, barcode_tensor, barcode_id, h, c, Wi, bi, Wh, bh,
           dnd_keys, dnd_vals, W1, b1, Wa, ba, Wc, bc):
    f32 = jnp.float32
    idc = barcode_id.reshape(B, 1)
    idr = barcode_id.reshape(1, B)

    vmem = lambda shape: pl.BlockSpec(shape, lambda g: (0, 0))
    tile = lambda w: pl.BlockSpec((TILE, w), lambda g: (g, 0))

    nk_pre, nv_pre, best_id, q_payload = pl.pallas_call(
        _k1_body,
        grid=(NT,),
        in_specs=[vmem((B, KDIM)), vmem((B, 1)), vmem((1, B)),
                  tile(KDIM), tile(H)],
        out_specs=[tile(KDIM), tile(H), vmem((B, 1)), vmem((B, KDIM))],
        out_shape=[jax.ShapeDtypeStruct((DICT, KDIM), f32),
                   jax.ShapeDtypeStruct((DICT, H), f32),
                   jax.ShapeDtypeStruct((B, 1), jnp.int32),
                   jax.ShapeDtypeStruct((B, KDIM), f32)],
        scratch_shapes=[pltpu.VMEM((B, 1), f32), pltpu.VMEM((B, 1), jnp.int32)],
    )(barcode_tensor, idc, idr, dnd_keys, dnd_vals)

    smem1d = pl.BlockSpec(memory_space=pltpu.SMEM)
    anyspec = pl.BlockSpec(memory_space=pl.ANY)
    vfull = pl.BlockSpec(memory_space=pltpu.VMEM)

    outs = pl.pallas_call(
        _k2_body,
        in_specs=[smem1d, smem1d, vfull, vfull, vfull, vfull, vfull,
                  vfull, vfull, vfull, vfull, vfull, vfull, vfull, smem1d,
                  vfull, vfull, vfull, anyspec, anyspec, anyspec],
        out_specs=[vfull, vfull, vfull, vfull, vfull, vfull, anyspec, anyspec],
        out_shape=[jax.ShapeDtypeStruct((B, 1), jnp.int32),
                   jax.ShapeDtypeStruct((B, 1), f32),
                   jax.ShapeDtypeStruct((B, 1), f32),
                   jax.ShapeDtypeStruct((B, 1), f32),
                   jax.ShapeDtypeStruct((B, H), f32),
                   jax.ShapeDtypeStruct((B, H), f32),
                   jax.ShapeDtypeStruct((DICT, KDIM), f32),
                   jax.ShapeDtypeStruct((DICT, H), f32)],
        scratch_shapes=[pltpu.VMEM((B, H), f32), pltpu.VMEM((B, H), f32),
                        pltpu.SemaphoreType.DMA],
        input_output_aliases={19: 6, 20: 7},
    )(best_id.reshape(B), barcode_id, obs_bar_reward, h, c,
      Wi, bi.reshape(1, 5 * H), Wh, bh.reshape(1, 5 * H),
      W1, b1.reshape(1, A2CH), Wa, ba.reshape(1, NA), Wc, bc,
      idc, idr, q_payload, dnd_vals, nk_pre, nv_pre)

    a_t, prob_a_t, v_t, entropy, h_t, c_t, new_keys, new_vals = outs
    return (a_t.reshape(B), prob_a_t.reshape(B), v_t, entropy.reshape(B),
            h_t, c_t, best_id.reshape(B), new_keys, new_vals)
